# Initial kernel scaffold; baseline (speedup 1.0000x reference)
import jax
import jax.numpy as jnp
from jax.experimental import pallas as pl


def kernel(features, W1, b1, W2, b2, W3, b3, edge_index):
    raise NotImplementedError("write your pallas kernel here")


# speedup vs baseline: 3.4589x; 3.4589x over previous
import functools

import jax
import jax.numpy as jnp
from jax import lax
from jax.experimental import pallas as pl
from jax.experimental.pallas import tpu as pltpu
from jax.experimental.pallas import tpu_sc as plsc

N = 10000
E = 160000

_NSC = 2
_NTILE = 16
_K = 80
_NPAD = 10240


def _make_degrees():
    ept = E // _NTILE
    nchunks = ept // _K
    mesh = plsc.VectorSubcoreMesh(core_axis_name="c", subcore_axis_name="s")

    @functools.partial(
        pl.kernel,
        out_type=(
            jax.ShapeDtypeStruct((N,), jnp.float32),
            jax.ShapeDtypeStruct((N,), jnp.float32),
        ),
        mesh=mesh,
        scratch_types=[
            pltpu.VMEM((_K,), jnp.int32),
            pltpu.VMEM((_K,), jnp.float32),
            pltpu.VMEM((640,), jnp.float32),
            pltpu.VMEM((1000,), jnp.float32),
            pltpu.VMEM_SHARED((_NPAD,), jnp.float32),
        ],
    )
    def deg_kernel(src_hbm, dst_hbm, dego_hbm, degi_hbm, idx_v, ones_v, zb_v,
                   stage_v, hist):
        c = lax.axis_index("c")
        s = lax.axis_index("s")
        for j in range(640 // 16):
            zb_v[pl.ds(j * 16, 16)] = jnp.zeros((16,), jnp.float32)
        for j in range(_K // 16):
            ones_v[pl.ds(j * 16, 16)] = jnp.ones((16,), jnp.float32)
        pltpu.sync_copy(zb_v, hist.at[pl.ds(s * 640, 640)])
        plsc.subcore_barrier()

        def run(idx_hbm):
            def chunk(g, carry):
                base = s * ept + g * _K
                pltpu.sync_copy(idx_hbm.at[pl.ds(base, _K)], idx_v)
                pltpu.sync_copy(ones_v, hist.at[idx_v], add=True)
                return carry

            lax.fori_loop(0, nchunks, chunk, 0)

        @pl.when(c == 0)
        def _():
            run(src_hbm)

        @pl.when(c == 1)
        def _():
            run(dst_hbm)

        plsc.subcore_barrier()

        @pl.when(jnp.logical_and(c == 0, s < 10))
        def _():
            pltpu.sync_copy(hist.at[pl.ds(s * 1000, 1000)], stage_v)
            pltpu.sync_copy(stage_v, dego_hbm.at[pl.ds(s * 1000, 1000)])

        @pl.when(jnp.logical_and(c == 1, s < 10))
        def _():
            pltpu.sync_copy(hist.at[pl.ds(s * 1000, 1000)], stage_v)
            pltpu.sync_copy(stage_v, degi_hbm.at[pl.ds(s * 1000, 1000)])

    return deg_kernel


def _make_agg(w):
    ept = E // _NTILE
    nchunks = ept // _K
    rpt = _NPAD // _NTILE
    mesh = plsc.VectorSubcoreMesh(core_axis_name="c", subcore_axis_name="s")

    @functools.partial(
        pl.kernel,
        out_type=jax.ShapeDtypeStruct((_NSC, _NPAD, w), jnp.float32),
        mesh=mesh,
        scratch_types=[
            pltpu.VMEM((_K,), jnp.int32),
            pltpu.VMEM((_K,), jnp.int32),
            pltpu.VMEM((_K, w), jnp.float32),
            pltpu.VMEM((32, w), jnp.float32),
            pltpu.VMEM_SHARED((_NPAD, w), jnp.float32),
            pltpu.SemaphoreType.DMA,
        ],
    )
    def agg_kernel(t_hbm, src_hbm, dst_hbm, out_hbm, sidx, didx, rows, zbuf,
                   acc, sem):
        c = lax.axis_index("c")
        s = lax.axis_index("s")
        for i in range(32):
            for j in range(w // 16):
                zbuf[i, pl.ds(j * 16, 16)] = jnp.zeros((16,), jnp.float32)

        def zb(i, carry):
            pltpu.sync_copy(zbuf, acc.at[pl.ds(s * rpt + i * 32, 32)])
            return carry

        lax.fori_loop(0, rpt // 32, zb, 0)
        plsc.subcore_barrier()

        coff = c * _NPAD

        def chunk(g, carry):
            base = s * ept + g * _K
            pltpu.sync_copy(src_hbm.at[pl.ds(base, _K)], sidx)
            pltpu.sync_copy(dst_hbm.at[pl.ds(base, _K)], didx)
            for j in range(_K // 16):
                sidx[pl.ds(j * 16, 16)] = sidx[pl.ds(j * 16, 16)] + coff
            pltpu.async_copy(t_hbm.at[sidx], rows, sem).wait()
            pltpu.sync_copy(rows, acc.at[didx], add=True)
            return carry

        lax.fori_loop(0, nchunks, chunk, 0)
        plsc.subcore_barrier()

        def ob(i, carry):
            r = s * rpt + i * 128
            pltpu.sync_copy(acc.at[pl.ds(r, 128)], out_hbm.at[c, pl.ds(r, 128)])
            return carry

        lax.fori_loop(0, rpt // 128, ob, 0)

    return agg_kernel


def _make_agg_sum(w):
    eps = E // _NSC
    ept = eps // _NTILE
    K = 40
    nchunks = ept // K
    rpt = _NPAD // _NTILE
    mesh = plsc.VectorSubcoreMesh(core_axis_name="c", subcore_axis_name="s")

    @functools.partial(
        pl.kernel,
        out_type=jax.ShapeDtypeStruct((_NSC, _NPAD, w), jnp.float32),
        mesh=mesh,
        scratch_types=[
            pltpu.VMEM((K,), jnp.int32),
            pltpu.VMEM((K,), jnp.int32),
            pltpu.VMEM((K, w), jnp.float32),
            pltpu.VMEM((32, w), jnp.float32),
            pltpu.VMEM_SHARED((_NPAD, w), jnp.float32),
            pltpu.SemaphoreType.DMA,
        ],
    )
    def agg_kernel(t_hbm, src_hbm, dst_hbm, out_hbm, sidx, didx, rows, zbuf,
                   acc, sem):
        c = lax.axis_index("c")
        s = lax.axis_index("s")
        for i in range(32):
            for j in range(w // 16):
                zbuf[i, pl.ds(j * 16, 16)] = jnp.zeros((16,), jnp.float32)

        def zb(i, carry):
            pltpu.sync_copy(zbuf, acc.at[pl.ds(s * rpt + i * 32, 32)])
            return carry

        lax.fori_loop(0, rpt // 32, zb, 0)
        plsc.subcore_barrier()

        def chunk(g, carry):
            base = c * eps + s * ept + g * K
            pltpu.sync_copy(src_hbm.at[pl.ds(base, K)], sidx)
            pltpu.sync_copy(dst_hbm.at[pl.ds(base, K)], didx)
            pltpu.async_copy(t_hbm.at[sidx], rows, sem).wait()
            pltpu.sync_copy(rows, acc.at[didx], add=True)
            return carry

        lax.fori_loop(0, nchunks, chunk, 0)
        plsc.subcore_barrier()

        def ob(i, carry):
            r = s * rpt + i * 128
            pltpu.sync_copy(acc.at[pl.ds(r, 128)], out_hbm.at[c, pl.ds(r, 128)])
            return carry

        lax.fori_loop(0, rpt // 128, ob, 0)

    return agg_kernel


_B = 1000


def _mm1_body(x_ref, w_ref, dego_ref, out_ref):
    ns = lax.rsqrt(jnp.maximum(dego_ref[...], 1.0))
    t = jnp.dot(x_ref[...], w_ref[...], preferred_element_type=jnp.float32)
    t = t * ns
    h = t.shape[1] // 2
    out_ref[0] = t[:, :h]
    out_ref[1] = t[:, h:]


def _mm1(x, w, dego):
    dout = w.shape[1]
    return pl.pallas_call(
        _mm1_body,
        grid=(N // _B,),
        in_specs=[
            pl.BlockSpec((_B, x.shape[1]), lambda i: (i, 0)),
            pl.BlockSpec(w.shape, lambda i: (0, 0)),
            pl.BlockSpec((_B, 1), lambda i: (i, 0)),
        ],
        out_specs=pl.BlockSpec((_NSC, _B, dout // 2), lambda i: (0, i, 0)),
        out_shape=jax.ShapeDtypeStruct((_NSC, _NPAD, dout // 2), jnp.float32),
    )(x, w, dego)


def _mm23_body(split, agg_ref, w_ref, b_ref, degi_ref, dego_ref, out_ref):
    nd = lax.rsqrt(jnp.maximum(degi_ref[...], 1.0))
    ns = lax.rsqrt(jnp.maximum(dego_ref[...], 1.0))
    m = (jnp.dot(agg_ref[0], w_ref[:128, :], preferred_element_type=jnp.float32)
         + jnp.dot(agg_ref[1], w_ref[128:, :], preferred_element_type=jnp.float32))
    bw = jnp.dot(b_ref[...], w_ref[...], preferred_element_type=jnp.float32)
    t = (m * nd + bw) * ns
    if split:
        h = t.shape[1] // 2
        out_ref[0] = t[:, :h]
        out_ref[1] = t[:, h:]
    else:
        out_ref[...] = t


def _mm23(agg, w, b_prev, degi, dego, split=True):
    dout = w.shape[1]
    if split:
        out_specs = pl.BlockSpec((_NSC, _B, dout // 2), lambda i: (0, i, 0))
        out_shape = jax.ShapeDtypeStruct((_NSC, _NPAD, dout // 2), jnp.float32)
    else:
        out_specs = pl.BlockSpec((_B, dout), lambda i: (i, 0))
        out_shape = jax.ShapeDtypeStruct((_NPAD, dout), jnp.float32)
    return pl.pallas_call(
        functools.partial(_mm23_body, split),
        grid=(N // _B,),
        in_specs=[
            pl.BlockSpec((_NSC, _B, 128), lambda i: (0, i, 0)),
            pl.BlockSpec(w.shape, lambda i: (0, 0)),
            pl.BlockSpec(b_prev.shape, lambda i: (0, 0)),
            pl.BlockSpec((_B, 1), lambda i: (i, 0)),
            pl.BlockSpec((_B, 1), lambda i: (i, 0)),
        ],
        out_specs=out_specs,
        out_shape=out_shape,
    )(agg, w, b_prev, degi, dego)


def _epi_body(agg_ref, degi_ref, b_ref, out_ref):
    nd = lax.rsqrt(jnp.maximum(degi_ref[...], 1.0))
    m = agg_ref[0] + agg_ref[1]
    out_ref[...] = m * nd + b_ref[...]


def _epi(agg, degi, b):
    dout = agg.shape[2]
    return pl.pallas_call(
        _epi_body,
        grid=(N // _B,),
        in_specs=[
            pl.BlockSpec((_NSC, _B, agg.shape[2]), lambda i: (0, i, 0)),
            pl.BlockSpec((_B, 1), lambda i: (i, 0)),
            pl.BlockSpec(b.shape, lambda i: (0, 0)),
        ],
        out_specs=pl.BlockSpec((_B, dout), lambda i: (i, 0)),
        out_shape=jax.ShapeDtypeStruct((N, dout), jnp.float32),
    )(agg, degi, b)


_make_degrees = functools.cache(_make_degrees)
_make_agg = functools.cache(_make_agg)
_make_agg_sum = functools.cache(_make_agg_sum)


def kernel(features, W1, b1, W2, b2, W3, b3, edge_index):
    src = edge_index[0]
    dst = edge_index[1]
    _deg_kernel = _make_degrees()
    _agg128 = _make_agg(128)
    _aggsum = _make_agg_sum(128)
    deg_out, deg_in = _deg_kernel(src, dst)
    dego = deg_out.reshape(N, 1)
    degi = deg_in.reshape(N, 1)

    t1 = _mm1(features, W1, dego)
    a1 = _agg128(t1.reshape(_NSC * _NPAD, 128), src, dst)
    t2 = _mm23(a1, W2, b1.reshape(1, -1), degi, dego)
    a2 = _agg128(t2.reshape(_NSC * _NPAD, 128), src, dst)
    t3 = _mm23(a2, W3, b2.reshape(1, -1), degi, dego,
               split=False)
    a3 = _aggsum(t3, src, dst)
    return _epi(a3, degi, b3.reshape(1, -1))


# speedup vs baseline: 9.5592x; 2.7636x over previous
import functools

import jax
import jax.numpy as jnp
from jax import lax
from jax.experimental import pallas as pl
from jax.experimental.pallas import tpu as pltpu
from jax.experimental.pallas import tpu_sc as plsc

N = 10000
E = 160000

_NSC = 2
_NTILE = 16
_K = 80
_NPAD = 10240


_CH = 112
_NCH = 90


def _make_degrees():
    mesh = plsc.VectorSubcoreMesh(core_axis_name="c", subcore_axis_name="s")

    @functools.partial(
        pl.kernel,
        out_type=(
            jax.ShapeDtypeStruct((N,), jnp.float32),
            jax.ShapeDtypeStruct((N,), jnp.float32),
        ),
        mesh=mesh,
        scratch_types=[
            pltpu.VMEM((_NCH, _CH), jnp.int32),
            pltpu.VMEM((_CH,), jnp.float32),
            pltpu.VMEM((640,), jnp.float32),
            pltpu.VMEM((1000,), jnp.float32),
            pltpu.VMEM_SHARED((_NPAD,), jnp.float32),
            pltpu.SemaphoreType.DMA,
        ],
    )
    def deg_kernel(idx_hbm, dego_hbm, degi_hbm, idx2, ones_v, zb_v,
                   stage_v, hist, sem):
        c = lax.axis_index("c")
        s = lax.axis_index("s")
        for j in range(640 // 16):
            zb_v[pl.ds(j * 16, 16)] = jnp.zeros((16,), jnp.float32)
        for j in range(_CH // 16):
            ones_v[pl.ds(j * 16, 16)] = jnp.ones((16,), jnp.float32)
        pltpu.sync_copy(idx_hbm.at[c, s], idx2)
        pltpu.sync_copy(zb_v, hist.at[pl.ds(s * 640, 640)])
        plsc.subcore_barrier()

        def batch(b, carry):
            for k in range(6):
                pltpu.async_copy(ones_v, hist.at[idx2.at[b * 6 + k]], sem,
                                 add=True)
            for k in range(6):
                pltpu.make_async_copy(ones_v, hist.at[idx2.at[b * 6 + k]],
                                      sem).wait()
            return carry

        lax.fori_loop(0, _NCH // 6, batch, 0)
        plsc.subcore_barrier()

        @pl.when(jnp.logical_and(c == 0, s < 10))
        def _():
            pltpu.sync_copy(hist.at[pl.ds(s * 1000, 1000)], stage_v)
            pltpu.sync_copy(stage_v, dego_hbm.at[pl.ds(s * 1000, 1000)])

        @pl.when(jnp.logical_and(c == 1, s < 10))
        def _():
            pltpu.sync_copy(hist.at[pl.ds(s * 1000, 1000)], stage_v)
            pltpu.sync_copy(stage_v, degi_hbm.at[pl.ds(s * 1000, 1000)])

    return deg_kernel


def _pipe(t_hbm, acc, idx_hbm, c, s, ibs, rows, gsems, ssems, nchunks):
    def i_load(j, k):
        pltpu.sync_copy(idx_hbm.at[c, s, j], ibs[k])

    def g_issue(k):
        pltpu.async_copy(t_hbm.at[ibs[k].at[0]], rows[k], gsems[k])

    def g_wait(k):
        pltpu.make_async_copy(t_hbm.at[ibs[k].at[0]], rows[k],
                              gsems[k]).wait()

    def s_issue(k):
        pltpu.async_copy(rows[k], acc.at[ibs[k].at[1]], ssems[k], add=True)

    def s_wait(k):
        pltpu.make_async_copy(rows[k], acc.at[ibs[k].at[1]],
                              ssems[k]).wait()

    nsl = len(ibs)
    niter = nchunks // nsl

    def prologue():
        for k in range(nsl - 1):
            i_load(k, k)
            g_issue(k)

    def body(i, carry):
        a = nsl * i

        @pl.when(i > 0)
        def _():
            s_wait(nsl - 1)

        i_load(a + nsl - 1, nsl - 1)
        g_issue(nsl - 1)
        g_wait(0)
        s_issue(0)
        g_wait(1)
        s_issue(1)
        for k in range(nsl - 2):
            s_wait(k)

            @pl.when(i < niter - 1)
            def _(k=k):
                i_load(a + nsl + k, k)
                g_issue(k)

            g_wait(k + 2)
            s_issue(k + 2)
        s_wait(nsl - 2)

        @pl.when(i < niter - 1)
        def _():
            i_load(a + 2 * nsl - 2, nsl - 2)
            g_issue(nsl - 2)

        return carry

    def main():
        lax.fori_loop(0, niter, body, 0)
        s_wait(nsl - 1)

    return prologue, main


def _zero_acc(rows0, acc, s, rpt, w):
    for i in range(_CH):
        for j in range(w // 16):
            rows0[i, pl.ds(j * 16, 16)] = jnp.zeros((16,), jnp.float32)

    def zb(i, carry):
        pltpu.sync_copy(rows0, acc.at[pl.ds(s * rpt + i * _CH, _CH)])
        return carry

    lax.fori_loop(0, rpt // _CH, zb, 0)
    rem = rpt % _CH
    if rem:
        pltpu.sync_copy(rows0.at[pl.ds(0, rem)],
                        acc.at[pl.ds(s * rpt + (rpt // _CH) * _CH, rem)])


def _make_agg(w, nchunks):
    rpt = _NPAD // _NTILE
    mesh = plsc.VectorSubcoreMesh(core_axis_name="c", subcore_axis_name="s")

    @functools.partial(
        pl.kernel,
        out_type=jax.ShapeDtypeStruct((_NSC, _NPAD, w), jnp.float32),
        mesh=mesh,
        scratch_types=[
            pltpu.VMEM((2, _CH), jnp.int32),
            pltpu.VMEM((2, _CH), jnp.int32),
            pltpu.VMEM((2, _CH), jnp.int32),
            pltpu.VMEM((_CH, w), jnp.float32),
            pltpu.VMEM((_CH, w), jnp.float32),
            pltpu.VMEM((_CH, w), jnp.float32),
            pltpu.VMEM_SHARED((_NPAD, w), jnp.float32),
            pltpu.SemaphoreType.DMA,
            pltpu.SemaphoreType.DMA,
            pltpu.SemaphoreType.DMA,
            pltpu.SemaphoreType.DMA,
            pltpu.SemaphoreType.DMA,
            pltpu.SemaphoreType.DMA,
        ],
    )
    def agg_kernel(t_hbm, sd_hbm, out_hbm, ib0, ib1, ib2, rows0, rows1,
                   rows2, acc, g0, g1, g2, s0, s1, s2):
        c = lax.axis_index("c")
        s = lax.axis_index("s")
        prologue, main = _pipe(t_hbm, acc, sd_hbm, c, s, [ib0, ib1, ib2],
                               [rows1, rows2, rows0], [g0, g1, g2],
                               [s0, s1, s2], nchunks)
        prologue()
        _zero_acc(rows0, acc, s, rpt, w)
        plsc.subcore_barrier()
        main()
        plsc.subcore_barrier()

        def ob(i, carry):
            r = s * rpt + i * 128
            pltpu.sync_copy(acc.at[pl.ds(r, 128)], out_hbm.at[c, pl.ds(r, 128)])
            return carry

        lax.fori_loop(0, rpt // 128, ob, 0)

    return agg_kernel


_B = 1000


def _mm1_body(x_ref, w_ref, dego_ref, out_ref):
    ns = lax.rsqrt(jnp.maximum(dego_ref[...], 1.0))
    t = jnp.dot(x_ref[...].astype(jnp.bfloat16), w_ref[...].astype(jnp.bfloat16),
                preferred_element_type=jnp.float32)
    t = t * ns
    h = t.shape[1] // 2
    out_ref[0] = t[:, :h]
    out_ref[1] = t[:, h:]


def _mm1(x, w, dego):
    dout = w.shape[1]
    return pl.pallas_call(
        _mm1_body,
        grid=(N // _B,),
        in_specs=[
            pl.BlockSpec((_B, x.shape[1]), lambda i: (i, 0)),
            pl.BlockSpec(w.shape, lambda i: (0, 0)),
            pl.BlockSpec((_B, 1), lambda i: (i, 0)),
        ],
        out_specs=pl.BlockSpec((_NSC, _B, dout // 2), lambda i: (0, i, 0)),
        out_shape=jax.ShapeDtypeStruct((_NSC, _NPAD, dout // 2), jnp.float32),
    )(x, w, dego)


def _mm23_body(split, agg_ref, w_ref, b_ref, degi_ref, dego_ref, out_ref):
    nd = lax.rsqrt(jnp.maximum(degi_ref[...], 1.0))
    ns = lax.rsqrt(jnp.maximum(dego_ref[...], 1.0))
    wb = w_ref[...].astype(jnp.bfloat16)
    m = (jnp.dot(agg_ref[0].astype(jnp.bfloat16), wb[:128, :],
                 preferred_element_type=jnp.float32)
         + jnp.dot(agg_ref[1].astype(jnp.bfloat16), wb[128:, :],
                   preferred_element_type=jnp.float32))
    bw = jnp.dot(b_ref[...], w_ref[...], preferred_element_type=jnp.float32)
    t = (m * nd + bw) * ns
    if split:
        h = t.shape[1] // 2
        out_ref[0] = t[:, :h]
        out_ref[1] = t[:, h:]
    else:
        out_ref[0] = t
        out_ref[1] = t


def _mm23(agg, w, b_prev, degi, dego, split=True):
    dout = w.shape[1]
    if split:
        out_specs = pl.BlockSpec((_NSC, _B, dout // 2), lambda i: (0, i, 0))
        out_shape = jax.ShapeDtypeStruct((_NSC, _NPAD, dout // 2), jnp.float32)
    else:
        out_specs = pl.BlockSpec((_NSC, _B, dout), lambda i: (0, i, 0))
        out_shape = jax.ShapeDtypeStruct((_NSC, _NPAD, dout), jnp.float32)
    return pl.pallas_call(
        functools.partial(_mm23_body, split),
        grid=(N // _B,),
        in_specs=[
            pl.BlockSpec((_NSC, _B, 128), lambda i: (0, i, 0)),
            pl.BlockSpec(w.shape, lambda i: (0, 0)),
            pl.BlockSpec(b_prev.shape, lambda i: (0, 0)),
            pl.BlockSpec((_B, 1), lambda i: (i, 0)),
            pl.BlockSpec((_B, 1), lambda i: (i, 0)),
        ],
        out_specs=out_specs,
        out_shape=out_shape,
    )(agg, w, b_prev, degi, dego)


def _epi_body(agg_ref, degi_ref, b_ref, out_ref):
    nd = lax.rsqrt(jnp.maximum(degi_ref[...], 1.0))
    m = agg_ref[0] + agg_ref[1]
    out_ref[...] = m * nd + b_ref[...]


def _epi(agg, degi, b):
    dout = agg.shape[2]
    return pl.pallas_call(
        _epi_body,
        grid=(N // _B,),
        in_specs=[
            pl.BlockSpec((_NSC, _B, agg.shape[2]), lambda i: (0, i, 0)),
            pl.BlockSpec((_B, 1), lambda i: (i, 0)),
            pl.BlockSpec(b.shape, lambda i: (0, 0)),
        ],
        out_specs=pl.BlockSpec((_B, dout), lambda i: (i, 0)),
        out_shape=jax.ShapeDtypeStruct((N, dout), jnp.float32),
    )(agg, degi, b)


_make_degrees = functools.cache(_make_degrees)
_make_agg = functools.cache(_make_agg)


def kernel(features, W1, b1, W2, b2, W3, b3, edge_index):
    src = edge_index[0]
    dst = edge_index[1]
    _deg_kernel = _make_degrees()
    _agg128 = _make_agg(128, _NCH)
    _aggsum = _make_agg(128, 48)

    def _pad_spread(x, total):
        padlen = total - x.shape[-1]
        tid = jnp.arange(x.shape[-2], dtype=jnp.int32)[:, None]
        padv = N + (tid * 53 + jnp.arange(padlen, dtype=jnp.int32)) % (_NPAD - N)
        padv = jnp.broadcast_to(padv.astype(jnp.int32),
                                x.shape[:-1] + (padlen,))
        return jnp.concatenate([x, padv], axis=-1)

    ept = E // _NTILE
    srcp = _pad_spread(src.reshape(_NTILE, ept),
                       _NCH * _CH).reshape(_NTILE, _NCH, _CH)
    dstp = _pad_spread(dst.reshape(_NTILE, ept),
                       _NCH * _CH).reshape(_NTILE, _NCH, _CH)
    deg_idx = jnp.stack([srcp, dstp])
    sdc = jnp.stack([jnp.stack([srcp, dstp], axis=2),
                     jnp.stack([srcp + _NPAD, dstp], axis=2)])
    epw = E // _NSC // _NTILE
    srcq = _pad_spread(src.reshape(_NSC, _NTILE, epw),
                       48 * _CH).reshape(_NSC, _NTILE, 48, _CH)
    srcq = srcq + jnp.arange(_NSC, dtype=jnp.int32).reshape(_NSC, 1, 1, 1) * _NPAD
    dstq = _pad_spread(dst.reshape(_NSC, _NTILE, epw),
                       48 * _CH).reshape(_NSC, _NTILE, 48, _CH)
    sdq = jnp.stack([srcq, dstq], axis=3)

    deg_out, deg_in = _deg_kernel(deg_idx)
    dego = deg_out.reshape(N, 1)
    degi = deg_in.reshape(N, 1)

    t1 = _mm1(features, W1, dego)
    a1 = _agg128(t1.reshape(_NSC * _NPAD, 128), sdc)
    t2 = _mm23(a1, W2, b1.reshape(1, -1), degi, dego)
    a2 = _agg128(t2.reshape(_NSC * _NPAD, 128), sdc)
    t3 = _mm23(a2, W3, b2.reshape(1, -1), degi, dego,
               split=False)
    a3 = _aggsum(t3.reshape(_NSC * _NPAD, 128), sdq)
    return _epi(a3, degi, b3.reshape(1, -1))
